# baseline (device time: 25626 ns/iter reference)
import jax
import jax.numpy as jnp
from jax import lax
from jax.experimental import pallas as pl
from jax.experimental.pallas import tpu as pltpu


def kernel(Q, K, V, bt, lens):
    B, _, H, D = Q.shape
    P_loc, BS = K.shape[0], K.shape[1]
    NB = bt.shape[1]
    KT = P_loc * BS
    HD = H * D
    HB = H * B
    scale = D ** -0.5

    Qt = Q.reshape(B, H, D).transpose(1, 0, 2) * scale
    Qbig = jnp.einsum("hbd,hg->hbgd", Qt, jnp.eye(H, dtype=Q.dtype))
    Qbig = Qbig.reshape(HB, HD)

    KT2 = K.transpose(1, 2, 3, 0).reshape(BS * HD, P_loc)
    VT2 = V.transpose(1, 2, 3, 0).reshape(BS * HD, P_loc)
    lens2 = lens.reshape(B, 1)

    def body(q_ref, k_ref, v_ref, bt_ref, lens_ref, out_ref,
             send_buf, recv_buf, send_sem, recv_sem):
        my_x = lax.axis_index("x")
        my_y = lax.axis_index("y")
        peer = (my_x, 1 - my_y)

        barrier = pltpu.get_barrier_semaphore()
        pl.semaphore_signal(barrier, inc=1, device_id=peer,
                            device_id_type=pl.DeviceIdType.MESH)
        pl.semaphore_wait(barrier, 1)

        valid = lax.broadcasted_iota(jnp.int32, (B, NB), 1) < lens_ref[:, :]
        pg = my_y * P_loc + lax.broadcasted_iota(jnp.int32, (P_loc, B, NB), 0)
        hit = (bt_ref[:, :][None, :, :] == pg) & valid[None, :, :]
        counts = jnp.sum(jnp.where(hit, 1.0, 0.0), axis=2)

        rb = lax.broadcasted_iota(jnp.int32, (B, HB), 1) % B
        bb = lax.broadcasted_iota(jnp.int32, (B, HB), 0)
        T = jnp.where(rb == bb, 1.0, 0.0)
        wP = lax.dot_general(T, counts, (((0,), (1,)), ((), ())),
                             preferred_element_type=jnp.float32)

        q = q_ref[:, :]
        obig = None
        psum = None
        for t in range(BS):
            k_t = k_ref[t * HD:(t + 1) * HD, :]
            v_t = v_ref[t * HD:(t + 1) * HD, :]
            s_t = lax.dot_general(q, k_t, (((1,), (0,)), ((), ())),
                                  preferred_element_type=jnp.float32)
            p_t = jnp.exp(s_t) * wP
            o_t = lax.dot_general(p_t, v_t, (((1,), (1,)), ((), ())),
                                  preferred_element_type=jnp.float32)
            obig = o_t if obig is None else obig + o_t
            psum = p_t if psum is None else psum + p_t
        lbig = lax.dot_general(psum, jnp.ones((P_loc, D), jnp.float32),
                               (((1,), (0,)), ((), ())),
                               preferred_element_type=jnp.float32)

        for h in range(H):
            sl = pl.ds(h * D, D)
            send_buf[:, sl] = obig[h * B:(h + 1) * B, h * D:(h + 1) * D]
            send_buf[:, pl.ds(HD + h * D, D)] = lbig[h * B:(h + 1) * B, :]

        rdma = pltpu.make_async_remote_copy(
            src_ref=send_buf, dst_ref=recv_buf,
            send_sem=send_sem, recv_sem=recv_sem,
            device_id=peer, device_id_type=pl.DeviceIdType.MESH)
        rdma.start()
        rdma.wait()

        o_tot = send_buf[:, :HD] + recv_buf[:, :HD]
        l_tot = send_buf[:, HD:] + recv_buf[:, HD:]
        out_ref[:, :] = o_tot / l_tot

    out = pl.pallas_call(
        body,
        out_shape=jax.ShapeDtypeStruct((B, HD), jnp.float32),
        in_specs=[pl.BlockSpec(memory_space=pltpu.VMEM)] * 5,
        out_specs=pl.BlockSpec(memory_space=pltpu.VMEM),
        scratch_shapes=[
            pltpu.VMEM((B, 2 * HD), jnp.float32),
            pltpu.VMEM((B, 2 * HD), jnp.float32),
            pltpu.SemaphoreType.DMA,
            pltpu.SemaphoreType.DMA,
        ],
        compiler_params=pltpu.CompilerParams(
            collective_id=0, vmem_limit_bytes=100 * 1024 * 1024),
    )(Qbig, KT2, VT2, bt, lens2)

    return out.reshape(B, 1, H, D)
